# baseline (speedup 1.0000x reference)
import jax
import jax.numpy as jnp
from jax import lax
from jax.experimental import pallas as pl
from jax.experimental.pallas import tpu as pltpu

_T = 4096
_D = 2048
_E = 64
_C = 64
_B = 256
_NBLK = _T // _B


def _gate_kernel(x_ref, w_ref, comb_ref, disp_ref, laux_ref, cnt_ref, gsum_ref):
    i = pl.program_id(0)

    @pl.when(i == 0)
    def _init():
        cnt_ref[...] = jnp.zeros_like(cnt_ref)
        gsum_ref[...] = jnp.zeros_like(gsum_ref)

    x = x_ref[...]
    w = w_ref[...]
    logits = lax.dot_general(
        x, w, (((1,), (1,)), ((), ())), preferred_element_type=jnp.float32
    )

    m = jnp.max(logits, axis=1, keepdims=True)
    ex = jnp.exp(logits - m)
    s = jnp.sum(ex, axis=1, keepdims=True)
    gates = ex / s
    gate1 = 1.0 / s

    eio = lax.broadcasted_iota(jnp.int32, (_B, _E), 1)
    is_max = logits == m
    idx = jnp.min(jnp.where(is_max, eio, _E), axis=1, keepdims=True)

    mask = (eio == idx).astype(jnp.float32)
    r = lax.broadcasted_iota(jnp.int32, (_B, _B), 0)
    c = lax.broadcasted_iota(jnp.int32, (_B, _B), 1)
    tri = (c <= r).astype(jnp.float32)
    csum = lax.dot_general(
        tri, mask, (((1,), (0,)), ((), ())), preferred_element_type=jnp.float32
    )

    carry = cnt_ref[...]
    loc_full = csum + carry - 1.0
    loc = jnp.sum(loc_full * mask, axis=1, keepdims=True)
    cnt_ref[...] = carry + csum[_B - 1 : _B, :]
    gsum_ref[...] = gsum_ref[...] + jnp.sum(gates, axis=0, keepdims=True)

    loc_i = loc.astype(jnp.int32)
    within = loc_i < _C
    locm = jnp.where(within, loc_i, -1)

    e3 = lax.broadcasted_iota(jnp.int32, (_B, _E, _C), 1)
    c3 = lax.broadcasted_iota(jnp.int32, (_B, _E, _C), 2)
    eq3 = (e3 == idx[:, :, None]) & (c3 == locm[:, :, None])
    comb_ref[...] = jnp.where(eq3, gate1[:, :, None], 0.0)
    disp_ref[...] = eq3

    @pl.when(i == _NBLK - 1)
    def _fin():
        cnt = cnt_ref[...]
        gs = gsum_ref[...]
        laux_ref[...] = jnp.sum(gs * cnt, axis=1, keepdims=True) * (_E / (_T * _T))


def kernel(input, W):
    comb_flat, disp_flat, laux = pl.pallas_call(
        _gate_kernel,
        grid=(_NBLK,),
        in_specs=[
            pl.BlockSpec((_B, _D), lambda i: (i, 0)),
            pl.BlockSpec((_E, _D), lambda i: (0, 0)),
        ],
        out_specs=[
            pl.BlockSpec((_B, _E, _C), lambda i: (i, 0, 0)),
            pl.BlockSpec((_B, _E, _C), lambda i: (i, 0, 0)),
            pl.BlockSpec((1, 1), lambda i: (0, 0)),
        ],
        out_shape=[
            jax.ShapeDtypeStruct((_T, _E, _C), jnp.float32),
            jax.ShapeDtypeStruct((_T, _E, _C), jnp.bool_),
            jax.ShapeDtypeStruct((1, 1), jnp.float32),
        ],
        scratch_shapes=[
            pltpu.VMEM((1, _E), jnp.float32),
            pltpu.VMEM((1, _E), jnp.float32),
        ],
    )(input, W)
    return laux[0, 0], comb_flat, disp_flat


# speedup vs baseline: 3.8130x; 3.8130x over previous
import jax
import jax.numpy as jnp
from jax import lax
from jax.experimental import pallas as pl
from jax.experimental.pallas import tpu as pltpu

_T = 4096
_D = 2048
_E = 64
_C = 64
_B = 256
_NBLK = _T // _B


def _gate_kernel(x_ref, w_ref, comb_ref, laux_ref, cnt_ref, gsum_ref):
    i = pl.program_id(0)

    @pl.when(i == 0)
    def _init():
        cnt_ref[...] = jnp.zeros_like(cnt_ref)
        gsum_ref[...] = jnp.zeros_like(gsum_ref)

    x = x_ref[...]
    w = w_ref[...]
    logits = lax.dot_general(
        w, x, (((1,), (1,)), ((), ())), preferred_element_type=jnp.float32
    )

    m = jnp.max(logits, axis=0, keepdims=True)
    ex = jnp.exp(logits - m)
    s = jnp.sum(ex, axis=0, keepdims=True)
    gates = ex / s
    gate1 = 1.0 / s

    eio = lax.broadcasted_iota(jnp.int32, (_E, _B), 0)
    idx = jnp.min(jnp.where(logits == m, eio, _E), axis=0, keepdims=True)

    mask = (eio == idx).astype(jnp.float32)
    r = lax.broadcasted_iota(jnp.int32, (_B, _B), 0)
    c = lax.broadcasted_iota(jnp.int32, (_B, _B), 1)
    tri = (r <= c).astype(jnp.float32)
    csum = lax.dot_general(
        mask, tri, (((1,), (0,)), ((), ())), preferred_element_type=jnp.float32
    )

    carry = cnt_ref[...]
    loc_full = csum + carry - 1.0
    loc = jnp.sum(loc_full * mask, axis=0, keepdims=True)
    cnt_ref[...] = carry + csum[:, _B - 1 : _B]
    gsum_ref[...] = gsum_ref[...] + jnp.sum(gates, axis=1, keepdims=True)

    loc_i = loc.astype(jnp.int32)
    locm = jnp.where(loc_i < _C, loc_i, -1)

    e3 = lax.broadcasted_iota(jnp.int32, (_E, _C, _B), 0)
    c3 = lax.broadcasted_iota(jnp.int32, (_E, _C, _B), 1)
    eq3 = (e3 == idx[None]) & (c3 == locm[None])
    comb_ref[...] = jnp.where(eq3, gate1[None], 0.0)

    @pl.when(i == _NBLK - 1)
    def _fin():
        cnt = cnt_ref[...]
        gs = gsum_ref[...]
        laux_ref[...] = jnp.sum(gs * cnt, axis=0, keepdims=True) * (_E / (_T * _T))


def kernel(input, W):
    comb_ect, laux = pl.pallas_call(
        _gate_kernel,
        grid=(_NBLK,),
        in_specs=[
            pl.BlockSpec((_B, _D), lambda i: (i, 0)),
            pl.BlockSpec((_E, _D), lambda i: (0, 0)),
        ],
        out_specs=[
            pl.BlockSpec((_E, _C, _B), lambda i: (0, 0, i)),
            pl.BlockSpec((1, 1), lambda i: (0, 0)),
        ],
        out_shape=[
            jax.ShapeDtypeStruct((_E, _C, _T), jnp.float32),
            jax.ShapeDtypeStruct((1, 1), jnp.float32),
        ],
        scratch_shapes=[
            pltpu.VMEM((_E, 1), jnp.float32),
            pltpu.VMEM((_E, 1), jnp.float32),
        ],
    )(input, W)
    combine = jnp.transpose(comb_ect, (2, 0, 1))
    dispatch = jnp.transpose(comb_ect != 0, (2, 0, 1))
    return laux[0, 0], combine, dispatch


# speedup vs baseline: 4.1423x; 1.0863x over previous
import jax
import jax.numpy as jnp
from jax import lax
from jax.experimental import pallas as pl
from jax.experimental.pallas import tpu as pltpu

_T = 4096
_D = 2048
_E = 64
_C = 64
_B = 256
_NBLK = _T // _B


def _gate_kernel(x_ref, w_ref, comb_ref, disp_ref, laux_ref, cnt_ref, gsum_ref):
    i = pl.program_id(0)

    @pl.when(i == 0)
    def _init():
        cnt_ref[...] = jnp.zeros_like(cnt_ref)
        gsum_ref[...] = jnp.zeros_like(gsum_ref)

    x = x_ref[...]
    w = w_ref[...]
    logits = lax.dot_general(
        w, x, (((1,), (1,)), ((), ())), preferred_element_type=jnp.float32
    )

    m = jnp.max(logits, axis=0, keepdims=True)
    ex = jnp.exp(logits - m)
    s = jnp.sum(ex, axis=0, keepdims=True)
    gates = ex / s
    gate1 = 1.0 / s

    eio = lax.broadcasted_iota(jnp.int32, (_E, _B), 0)
    idx = jnp.min(jnp.where(logits == m, eio, _E), axis=0, keepdims=True)

    mask = (eio == idx).astype(jnp.float32)
    r = lax.broadcasted_iota(jnp.int32, (_B, _B), 0)
    c = lax.broadcasted_iota(jnp.int32, (_B, _B), 1)
    tri = (r <= c).astype(jnp.float32)
    csum = lax.dot_general(
        mask, tri, (((1,), (0,)), ((), ())), preferred_element_type=jnp.float32
    )

    carry = cnt_ref[...]
    loc_full = csum + carry - 1.0
    loc = jnp.sum(loc_full * mask, axis=0, keepdims=True)
    cnt_ref[...] = carry + csum[:, _B - 1 : _B]
    gsum_ref[...] = gsum_ref[...] + jnp.sum(gates, axis=1, keepdims=True)

    loc_i = loc.astype(jnp.int32)
    locm = jnp.where(loc_i < _C, loc_i, -1)

    e3 = lax.broadcasted_iota(jnp.int32, (_E, _C, _B), 0)
    c3 = lax.broadcasted_iota(jnp.int32, (_E, _C, _B), 1)
    eq3 = (e3 == idx[None]) & (c3 == locm[None])
    comb_ref[...] = jnp.where(eq3, gate1[None], 0.0)
    disp_ref[...] = eq3.astype(jnp.int8)

    @pl.when(i == _NBLK - 1)
    def _fin():
        cnt = cnt_ref[...]
        gs = gsum_ref[...]
        laux_ref[...] = jnp.sum(gs * cnt, axis=0, keepdims=True) * (_E / (_T * _T))


def kernel(input, W):
    comb_ect, disp_ect, laux = pl.pallas_call(
        _gate_kernel,
        grid=(_NBLK,),
        in_specs=[
            pl.BlockSpec((_B, _D), lambda i: (i, 0)),
            pl.BlockSpec((_E, _D), lambda i: (0, 0)),
        ],
        out_specs=[
            pl.BlockSpec((_E, _C, _B), lambda i: (0, 0, i)),
            pl.BlockSpec((_E, _C, _B), lambda i: (0, 0, i)),
            pl.BlockSpec((1, 1), lambda i: (0, 0)),
        ],
        out_shape=[
            jax.ShapeDtypeStruct((_E, _C, _T), jnp.float32),
            jax.ShapeDtypeStruct((_E, _C, _T), jnp.int8),
            jax.ShapeDtypeStruct((1, 1), jnp.float32),
        ],
        scratch_shapes=[
            pltpu.VMEM((_E, 1), jnp.float32),
            pltpu.VMEM((_E, 1), jnp.float32),
        ],
    )(input, W)
    combine = jnp.transpose(comb_ect, (2, 0, 1))
    dispatch = jnp.transpose(disp_ect.view(jnp.bool_), (2, 0, 1))
    return laux[0, 0], combine, dispatch


# speedup vs baseline: 4.6174x; 1.1147x over previous
import jax
import jax.numpy as jnp
from jax import lax
from jax.experimental import pallas as pl
from jax.experimental.pallas import tpu as pltpu

_T = 4096
_D = 2048
_E = 64
_C = 64
_B = 256
_NBLK = _T // _B


def _gate_kernel(x_ref, w_ref, comb_ref, disp_ref, laux_ref, cnt_ref, gsum_ref):
    i = pl.program_id(0)

    @pl.when(i == 0)
    def _init():
        cnt_ref[...] = jnp.zeros_like(cnt_ref)
        gsum_ref[...] = jnp.zeros_like(gsum_ref)

    x = x_ref[...]
    w = w_ref[...]
    logits = lax.dot_general(
        w, x, (((1,), (1,)), ((), ())), preferred_element_type=jnp.float32
    )

    m = jnp.max(logits, axis=0, keepdims=True)
    ex = jnp.exp(logits - m)
    s = jnp.sum(ex, axis=0, keepdims=True)
    gates = ex / s
    gate1 = 1.0 / s

    eio = lax.broadcasted_iota(jnp.int32, (_E, _B), 0)
    idx = jnp.min(jnp.where(logits == m, eio, _E), axis=0, keepdims=True)

    mask = (eio == idx).astype(jnp.float32)
    r = lax.broadcasted_iota(jnp.int32, (_B, _B), 0)
    c = lax.broadcasted_iota(jnp.int32, (_B, _B), 1)
    tri = (r <= c).astype(jnp.float32)
    csum = lax.dot_general(
        mask, tri, (((1,), (0,)), ((), ())), preferred_element_type=jnp.float32
    )

    carry = cnt_ref[...]
    loc_full = csum + carry - 1.0
    loc = jnp.sum(loc_full * mask, axis=0, keepdims=True)
    cnt_ref[...] = carry + csum[:, _B - 1 : _B]
    gsum_ref[...] = gsum_ref[...] + jnp.sum(gates, axis=1, keepdims=True)

    loc_i = loc.astype(jnp.int32)
    locm = jnp.where(loc_i < _C, loc_i, -1)

    c2 = lax.broadcasted_iota(jnp.int32, (_C, _B), 0)
    onehot_c = c2 == locm
    valf = jnp.where(onehot_c, gate1, 0.0)
    vali = onehot_c.astype(jnp.int8)
    me = (eio == idx)[:, None, :]
    comb_ref[...] = jnp.where(me, valf[None], 0.0)
    disp_ref[...] = jnp.where(me, vali[None], jnp.int8(0))

    @pl.when(i == _NBLK - 1)
    def _fin():
        cnt = cnt_ref[...]
        gs = gsum_ref[...]
        laux_ref[...] = jnp.sum(gs * cnt, axis=0, keepdims=True) * (_E / (_T * _T))


def kernel(input, W):
    comb_ect, disp_ect, laux = pl.pallas_call(
        _gate_kernel,
        grid=(_NBLK,),
        in_specs=[
            pl.BlockSpec((_B, _D), lambda i: (i, 0)),
            pl.BlockSpec((_E, _D), lambda i: (0, 0)),
        ],
        out_specs=[
            pl.BlockSpec((_E, _C, _B), lambda i: (0, 0, i)),
            pl.BlockSpec((_E, _C, _B), lambda i: (0, 0, i)),
            pl.BlockSpec((1, 1), lambda i: (0, 0)),
        ],
        out_shape=[
            jax.ShapeDtypeStruct((_E, _C, _T), jnp.float32),
            jax.ShapeDtypeStruct((_E, _C, _T), jnp.int8),
            jax.ShapeDtypeStruct((1, 1), jnp.float32),
        ],
        scratch_shapes=[
            pltpu.VMEM((_E, 1), jnp.float32),
            pltpu.VMEM((_E, 1), jnp.float32),
        ],
    )(input, W)
    combine = jnp.transpose(comb_ect, (2, 0, 1))
    dispatch = jnp.transpose(disp_ect.view(jnp.bool_), (2, 0, 1))
    return laux[0, 0], combine, dispatch


# speedup vs baseline: 4.6828x; 1.0142x over previous
import jax
import jax.numpy as jnp
from jax import lax
from jax.experimental import pallas as pl
from jax.experimental.pallas import tpu as pltpu

_T = 4096
_D = 2048
_E = 64
_C = 64
_B = 512
_NBLK = _T // _B


def _gate_kernel(x_ref, w_ref, comb_ref, disp_ref, laux_ref, cnt_ref, gsum_ref):
    i = pl.program_id(0)

    @pl.when(i == 0)
    def _init():
        cnt_ref[...] = jnp.zeros_like(cnt_ref)
        gsum_ref[...] = jnp.zeros_like(gsum_ref)

    x = x_ref[...]
    w = w_ref[...]
    logits = lax.dot_general(
        w, x, (((1,), (1,)), ((), ())), preferred_element_type=jnp.float32
    )

    m = jnp.max(logits, axis=0, keepdims=True)
    ex = jnp.exp(logits - m)
    s = jnp.sum(ex, axis=0, keepdims=True)
    gates = ex / s
    gate1 = 1.0 / s

    eio = lax.broadcasted_iota(jnp.int32, (_E, _B), 0)
    idx = jnp.min(jnp.where(logits == m, eio, _E), axis=0, keepdims=True)

    mask = (eio == idx).astype(jnp.float32)
    r = lax.broadcasted_iota(jnp.int32, (_B, _B), 0)
    c = lax.broadcasted_iota(jnp.int32, (_B, _B), 1)
    tri = (r <= c).astype(jnp.float32)
    csum = lax.dot_general(
        mask, tri, (((1,), (0,)), ((), ())), preferred_element_type=jnp.float32
    )

    carry = cnt_ref[...]
    loc_full = csum + carry - 1.0
    loc = jnp.sum(loc_full * mask, axis=0, keepdims=True)
    cnt_ref[...] = carry + csum[:, _B - 1 : _B]
    gsum_ref[...] = gsum_ref[...] + jnp.sum(gates, axis=1, keepdims=True)

    loc_i = loc.astype(jnp.int32)
    locm = jnp.where(loc_i < _C, loc_i, -1)

    c2 = lax.broadcasted_iota(jnp.int32, (_C, _B), 0)
    onehot_c = c2 == locm
    valf = jnp.where(onehot_c, gate1, 0.0)
    vali = onehot_c.astype(jnp.int8)
    me = (eio == idx)[:, None, :]
    comb_ref[...] = jnp.where(me, valf[None], 0.0)
    disp_ref[...] = jnp.where(me, vali[None], jnp.int8(0))

    @pl.when(i == _NBLK - 1)
    def _fin():
        cnt = cnt_ref[...]
        gs = gsum_ref[...]
        laux_ref[...] = jnp.sum(gs * cnt, axis=0, keepdims=True) * (_E / (_T * _T))


def kernel(input, W):
    comb_ect, disp_ect, laux = pl.pallas_call(
        _gate_kernel,
        grid=(_NBLK,),
        in_specs=[
            pl.BlockSpec((_B, _D), lambda i: (i, 0)),
            pl.BlockSpec((_E, _D), lambda i: (0, 0)),
        ],
        out_specs=[
            pl.BlockSpec((_E, _C, _B), lambda i: (0, 0, i)),
            pl.BlockSpec((_E, _C, _B), lambda i: (0, 0, i)),
            pl.BlockSpec((1, 1), lambda i: (0, 0)),
        ],
        out_shape=[
            jax.ShapeDtypeStruct((_E, _C, _T), jnp.float32),
            jax.ShapeDtypeStruct((_E, _C, _T), jnp.int8),
            jax.ShapeDtypeStruct((1, 1), jnp.float32),
        ],
        scratch_shapes=[
            pltpu.VMEM((_E, 1), jnp.float32),
            pltpu.VMEM((_E, 1), jnp.float32),
        ],
    )(input, W)
    combine = jnp.transpose(comb_ect, (2, 0, 1))
    dispatch = jnp.transpose(disp_ect.view(jnp.bool_), (2, 0, 1))
    return laux[0, 0], combine, dispatch
